# baseline (speedup 1.0000x reference)
import functools

import jax
import jax.numpy as jnp
from jax import lax
from jax.experimental import pallas as pl
from jax.experimental.pallas import tpu as pltpu
from jax.experimental.pallas import tpu_sc as plsc

_B, _T, _D = 16384, 100, 64
_F = _T * _D
_LANES = 128
_MROWS = (_B * _T) // _LANES
_MBLK = 1600

_NC, _NS = 2, 16
_NW = _NC * _NS
_RPW = _B // _NW


@functools.partial(
    pl.kernel,
    out_type=jax.ShapeDtypeStruct((_B, _F), jnp.float32),
    mesh=plsc.VectorSubcoreMesh(core_axis_name="c", subcore_axis_name="s"),
    scratch_types=[pltpu.SemaphoreType.DMA],
)
def _sc_copy(x_hbm, y_hbm, sem):
    wid = lax.axis_index("s") * _NC + lax.axis_index("c")
    base = wid * _RPW
    pltpu.async_copy(
        x_hbm.at[pl.ds(base, _RPW)], y_hbm.at[pl.ds(base, _RPW)], sem
    ).wait()


def _zeros_mask_kernel(o_ref):
    o_ref[...] = jnp.zeros_like(o_ref)


def kernel(inputs):
    x2 = inputs.reshape(_B, _F)
    y2 = _sc_copy(x2)
    mask2d = pl.pallas_call(
        _zeros_mask_kernel,
        out_shape=jax.ShapeDtypeStruct((_MROWS, _LANES), inputs.dtype),
        grid=(_MROWS // _MBLK,),
        out_specs=pl.BlockSpec((_MBLK, _LANES), lambda i: (i, 0)),
    )()
    return (y2.reshape(_B, _T, _D), mask2d.reshape(_B, _T, 1))


# speedup vs baseline: 12.7349x; 12.7349x over previous
import functools

import jax
import jax.numpy as jnp
from jax import lax
from jax.experimental import pallas as pl
from jax.experimental.pallas import tpu as pltpu
from jax.experimental.pallas import tpu_sc as plsc

_B, _T, _D = 16384, 100, 64
_F = _T * _D
_LANES = 128
_MROWS = (_B * _T) // _LANES
_MBLK = 1600

_NC, _NS = 2, 16
_NW = _NC * _NS
_RPW = _B // _NW
_CR = 8
_NCH = _RPW // _CR


@functools.partial(
    pl.kernel,
    out_type=jax.ShapeDtypeStruct((_B, _F), jnp.float32),
    mesh=plsc.VectorSubcoreMesh(core_axis_name="c", subcore_axis_name="s"),
    scratch_types=[
        pltpu.VMEM((2, _CR, _F), jnp.float32),
        pltpu.SemaphoreType.DMA((2,)),
        pltpu.SemaphoreType.DMA((2,)),
    ],
)
def _sc_copy(x_hbm, y_hbm, bufs, sin, sout):
    wid = lax.axis_index("s") * _NC + lax.axis_index("c")
    base = wid * _RPW

    def in_copy(c, s):
        return pltpu.make_async_copy(
            x_hbm.at[pl.ds(base + c * _CR, _CR)], bufs.at[s], sin.at[s])

    def out_copy(c, s):
        return pltpu.make_async_copy(
            bufs.at[s], y_hbm.at[pl.ds(base + c * _CR, _CR)], sout.at[s])

    in_copy(0, 0).start()
    for c in range(_NCH):
        s = c % 2
        in_copy(c, s).wait()
        nxt = c + 1
        if nxt < _NCH:
            s2 = nxt % 2
            if c >= 1:
                out_copy(c - 1, s2).wait()
            in_copy(nxt, s2).start()
        out_copy(c, s).start()
    out_copy(_NCH - 1, (_NCH - 1) % 2).wait()


def _zeros_mask_kernel(o_ref):
    o_ref[...] = jnp.zeros_like(o_ref)


def kernel(inputs):
    x2 = inputs.reshape(_B, _F)
    y2 = _sc_copy(x2)
    mask2d = pl.pallas_call(
        _zeros_mask_kernel,
        out_shape=jax.ShapeDtypeStruct((_MROWS, _LANES), inputs.dtype),
        grid=(_MROWS // _MBLK,),
        out_specs=pl.BlockSpec((_MBLK, _LANES), lambda i: (i, 0)),
    )()
    return (y2.reshape(_B, _T, _D), mask2d.reshape(_B, _T, 1))


# speedup vs baseline: 43.6447x; 3.4272x over previous
import functools

import jax
import jax.numpy as jnp
from jax import lax
from jax.experimental import pallas as pl
from jax.experimental.pallas import tpu as pltpu
from jax.experimental.pallas import tpu_sc as plsc

_B, _T, _D = 16384, 100, 64
_N = _B * _T

_NC, _NS = 2, 16
_NW = _NC * _NS
_EPW = _N // _NW
_BUF = 6400
_REP = _EPW // _BUF
_VEC = 16


@functools.partial(
    pl.kernel,
    out_type=jax.ShapeDtypeStruct((_N,), jnp.float32),
    mesh=plsc.VectorSubcoreMesh(core_axis_name="c", subcore_axis_name="s"),
    scratch_types=[
        pltpu.VMEM((_BUF,), jnp.float32),
        pltpu.SemaphoreType.DMA,
    ],
)
def _sc_mask(mask_hbm, buf, sem):
    wid = lax.axis_index("s") * _NC + lax.axis_index("c")
    base = wid * _EPW
    zvec = jnp.zeros((_VEC,), jnp.float32)

    @pl.loop(0, _BUF // _VEC)
    def _zero(i):
        buf[pl.ds(i * _VEC, _VEC)] = zvec

    for k in range(_REP):
        pltpu.make_async_copy(
            buf, mask_hbm.at[pl.ds(base + k * _BUF, _BUF)], sem
        ).start()
    for k in range(_REP):
        pltpu.make_async_copy(
            buf, mask_hbm.at[pl.ds(base + k * _BUF, _BUF)], sem
        ).wait()


def kernel(inputs):
    mask = _sc_mask()
    return (inputs, mask.reshape(_B, _T, 1))
